# baseline (speedup 1.0000x reference)
import jax
import jax.numpy as jnp
from jax.experimental import pallas as pl
from jax.experimental.pallas import tpu as pltpu


def _ffn_body(e, thr, x32_ref, sim_ref, scores_ref, k_ref, w1_ref, w2_ref,
              eof_ref, k0_ref):
    @pl.when(e == 0)
    def _():
        xf = x32_ref[...]
        xn = xf / (jnp.sqrt(jnp.sum(xf * xf, axis=1, keepdims=True)) + 1e-12)
        sm = sim_ref[...]
        wn = sm / (jnp.sqrt(jnp.sum(sm * sm, axis=0, keepdims=True)) + 1e-12)
        s = jnp.dot(xn, wn, preferred_element_type=jnp.float32)
        scores_ref[...] = s
        kk = jnp.sum((s > thr).astype(jnp.int32), axis=1, keepdims=True)
        if k0_ref is None:
            k_ref[...] = kk
        else:
            k_ref[...] = k0_ref[...] + kk

    s_half = scores_ref[...]
    onehot = (jax.lax.broadcasted_iota(jnp.int32, s_half.shape, 1) == e)
    mask_col = jnp.sum(
        jnp.where((s_half > thr) & onehot, 1.0, 0.0), axis=1, keepdims=True)

    xb = x32_ref[...].astype(jnp.bfloat16)
    w1e = w1_ref[e]
    w2e = w2_ref[e]
    h = jax.lax.dot_general(xb, w1e, (((1,), (1,)), ((), ())),
                            preferred_element_type=jnp.float32)
    g = 0.5 * h * (1.0 + jax.lax.erf(h * 0.7071067811865476))
    out = jax.lax.dot_general(g.astype(jnp.bfloat16), w2e,
                              (((1,), (1,)), ((), ())),
                              preferred_element_type=jnp.float32)
    mo = out * mask_col
    eof_ref[:, 0, e, :] = mo
    return mo


def _step_first(x32_ref, sim_ref, thr_ref, w1_ref, w2_ref,
                final_ref, scores_ref, eof_ref, k_ref):
    e = pl.program_id(1)
    mo = _ffn_body(e, thr_ref[0, 0], x32_ref, sim_ref, scores_ref, k_ref,
                   w1_ref, w2_ref, eof_ref, None)

    @pl.when(e == 0)
    def _():
        final_ref[...] = mo

    @pl.when(e != 0)
    def _():
        final_ref[...] += mo


def _step_second(x32_ref, sim_ref, thr_ref, w1_ref, w2_ref, fin0_ref, k0_ref,
                 _eof_alias_ref,
                 final_ref, scores_ref, eof_ref, k_ref):
    e = pl.program_id(1)
    mo = _ffn_body(e, thr_ref[0, 0], x32_ref, sim_ref, scores_ref, k_ref,
                   w1_ref, w2_ref, eof_ref, k0_ref)

    @pl.when(e == 0)
    def _():
        final_ref[...] = fin0_ref[...] + mo

    @pl.when(e != 0)
    def _():
        final_ref[...] += mo


def _half_call(half, x32, sim_half, thr, w1b, w2b, prev, N, Cc, Ee, Dff, TB):
    NB = N // TB
    EH = Ee // 2
    grid = (NB, EH)
    out_shapes = (
        jax.ShapeDtypeStruct((N, Cc), jnp.float32),
        jax.ShapeDtypeStruct((N, EH), jnp.float32),
        jax.ShapeDtypeStruct((N, 2, EH, Cc), jnp.float32),
        jax.ShapeDtypeStruct((N, 1), jnp.int32),
    )
    in_specs = [
        pl.BlockSpec((TB, Cc), lambda n, e: (n, 0)),
        pl.BlockSpec((Cc, EH), lambda n, e: (0, 0)),
        pl.BlockSpec((1, 1), lambda n, e: (0, 0)),
        pl.BlockSpec((EH, Dff, Cc), lambda n, e: (half, 0, 0)),
        pl.BlockSpec((EH, Cc, Dff), lambda n, e: (half, 0, 0)),
    ]
    out_specs = (
        pl.BlockSpec((TB, Cc), lambda n, e: (n, 0)),
        pl.BlockSpec((TB, EH), lambda n, e: (n, 0)),
        pl.BlockSpec((TB, 1, EH, Cc), lambda n, e: (n, half, 0, 0)),
        pl.BlockSpec((TB, 1), lambda n, e: (n, 0)),
    )
    operands = [x32, sim_half, thr, w1b, w2b]
    if prev is None:
        body = _step_first
        aliases = {}
    else:
        fin0, eof0, k0 = prev
        body = _step_second
        in_specs += [
            pl.BlockSpec((TB, Cc), lambda n, e: (n, 0)),
            pl.BlockSpec((TB, 1), lambda n, e: (n, 0)),
            pl.BlockSpec((8, 1, EH, 128), lambda n, e: (0, 0, 0, 0)),
        ]
        operands += [fin0, k0, eof0]
        aliases = {7: 2}
    return pl.pallas_call(
        body,
        grid=grid,
        in_specs=in_specs,
        out_specs=out_specs,
        out_shape=out_shapes,
        input_output_aliases=aliases,
        compiler_params=pltpu.CompilerParams(
            dimension_semantics=("arbitrary", "arbitrary"),
            vmem_limit_bytes=63 * 1024 * 1024,
        ),
    )(*operands)


def kernel(hidden_states, sim_matrix, threshold, w1, w2):
    Bb, Tt, Cc = hidden_states.shape
    Ee, Dff, _ = w1.shape
    N = Bb * Tt
    EH = Ee // 2
    TB = 256

    x32 = hidden_states.reshape(N, Cc)
    w1b = w1.astype(jnp.bfloat16)
    w2b = w2.astype(jnp.bfloat16)
    thr = threshold.reshape(1, 1)
    sim_a = jax.lax.slice_in_dim(sim_matrix, 0, EH, axis=1)
    sim_b = jax.lax.slice_in_dim(sim_matrix, EH, Ee, axis=1)

    fin1, s_a, eof1, k1 = _half_call(0, x32, sim_a, thr, w1b, w2b, None,
                                     N, Cc, Ee, Dff, TB)
    final, s_b, eof4, k = _half_call(1, x32, sim_b, thr, w1b, w2b,
                                     (fin1, eof1, k1), N, Cc, Ee, Dff, TB)

    scores = jnp.concatenate([s_a, s_b], axis=1)
    eof = eof4.reshape(N, Ee, Cc)
    return (final.reshape(Bb, Tt, Cc), scores, eof, k.reshape(N))


# speedup vs baseline: 1.9380x; 1.9380x over previous
import jax
import jax.numpy as jnp
from jax.experimental import pallas as pl
from jax.experimental.pallas import tpu as pltpu


def _moe_step(x_ref, sim_ref, thr_ref, w1_hbm, w2_hbm,
              final_ref, scores_ref, eof_hbm, k_ref,
              stag1, stag2, wb1, wb2, eofscr, sem_w, sem_o):
    e = pl.program_id(0)
    n = pl.program_id(1)
    ne = pl.num_programs(0)
    nb = pl.num_programs(1)
    tb = eofscr.shape[0]
    thr = thr_ref[0, 0]
    step = e * nb + n

    @pl.when(step == 0)
    def _():
        xf = x_ref[...]
        xn = xf / (jnp.sqrt(jnp.sum(xf * xf, axis=1, keepdims=True)) + 1e-12)
        sm = sim_ref[...]
        wn = sm / (jnp.sqrt(jnp.sum(sm * sm, axis=0, keepdims=True)) + 1e-12)
        s = jnp.dot(xn, wn, preferred_element_type=jnp.float32)
        scores_ref[...] = s
        k_ref[...] = jnp.sum((s > thr).astype(jnp.int32), axis=1, keepdims=True)
        pltpu.make_async_copy(w1_hbm.at[0], stag1, sem_w).start()
        pltpu.make_async_copy(w2_hbm.at[0], stag2, sem_w).start()
        pltpu.make_async_copy(w1_hbm.at[0], stag1, sem_w).wait()
        pltpu.make_async_copy(w2_hbm.at[0], stag2, sem_w).wait()

    @pl.when(n == 0)
    def _():
        slot = jax.lax.rem(e, 2)

        @pl.when(e > 0)
        def _():
            pltpu.make_async_copy(w1_hbm.at[e], stag1, sem_w).wait()
            pltpu.make_async_copy(w2_hbm.at[e], stag2, sem_w).wait()

        wb1[slot] = stag1[...].astype(jnp.bfloat16)
        wb2[slot] = stag2[...].astype(jnp.bfloat16)

        @pl.when(e + 1 < ne)
        def _():
            pltpu.make_async_copy(w1_hbm.at[e + 1], stag1, sem_w).start()
            pltpu.make_async_copy(w2_hbm.at[e + 1], stag2, sem_w).start()

    slot = jax.lax.rem(e, 2)
    rows = pl.ds(n * tb, tb)
    xb = x_ref[rows, :].astype(jnp.bfloat16)
    h = jax.lax.dot_general(xb, wb1[slot], (((1,), (1,)), ((), ())),
                            preferred_element_type=jnp.float32)
    g = 0.5 * h * (1.0 + jax.lax.erf(h * 0.7071067811865476))
    out = jax.lax.dot_general(g.astype(jnp.bfloat16), wb2[slot],
                              (((1,), (1,)), ((), ())),
                              preferred_element_type=jnp.float32)
    s_blk = scores_ref[rows, :]
    onehot = (jax.lax.broadcasted_iota(jnp.int32, s_blk.shape, 1) == e)
    mask_col = jnp.sum(
        jnp.where((s_blk > thr) & onehot, 1.0, 0.0), axis=1, keepdims=True)
    mo = out * mask_col

    @pl.when(e == 0)
    def _():
        final_ref[rows, :] = mo

    @pl.when(e != 0)
    def _():
        final_ref[rows, :] = final_ref[rows, :] + mo

    @pl.when(step > 0)
    def _():
        sp = step - 1
        ep = sp // nb
        np_ = jax.lax.rem(sp, nb)
        prev_dst = eof_hbm.at[pl.ds(np_ * tb, tb), ep, :]
        pltpu.make_async_copy(eofscr, prev_dst, sem_o).wait()

    eofscr[...] = mo
    dst = eof_hbm.at[rows, e, :]
    pltpu.make_async_copy(eofscr, dst, sem_o).start()

    @pl.when(step == ne * nb - 1)
    def _():
        pltpu.make_async_copy(eofscr, dst, sem_o).wait()


def kernel(hidden_states, sim_matrix, threshold, w1, w2):
    Bb, Tt, Cc = hidden_states.shape
    Ee, Dff, _ = w1.shape
    N = Bb * Tt
    TB = 512
    NB = N // TB

    x32 = hidden_states.reshape(N, Cc)
    thr = threshold.reshape(1, 1)

    grid = (Ee, NB)
    out_shapes = (
        jax.ShapeDtypeStruct((N, Cc), jnp.float32),
        jax.ShapeDtypeStruct((N, Ee), jnp.float32),
        jax.ShapeDtypeStruct((N, Ee, Cc), jnp.float32),
        jax.ShapeDtypeStruct((N, 1), jnp.int32),
    )
    in_specs = [
        pl.BlockSpec((N, Cc), lambda e, n: (0, 0)),
        pl.BlockSpec((Cc, Ee), lambda e, n: (0, 0)),
        pl.BlockSpec((1, 1), lambda e, n: (0, 0)),
        pl.BlockSpec(memory_space=pltpu.MemorySpace.HBM),
        pl.BlockSpec(memory_space=pltpu.MemorySpace.HBM),
    ]
    out_specs = (
        pl.BlockSpec((N, Cc), lambda e, n: (0, 0)),
        pl.BlockSpec((N, Ee), lambda e, n: (0, 0)),
        pl.BlockSpec(memory_space=pltpu.MemorySpace.HBM),
        pl.BlockSpec((N, 1), lambda e, n: (0, 0)),
    )
    scratch_shapes = [
        pltpu.VMEM((Dff, Cc), jnp.float32),
        pltpu.VMEM((Cc, Dff), jnp.float32),
        pltpu.VMEM((2, Dff, Cc), jnp.bfloat16),
        pltpu.VMEM((2, Cc, Dff), jnp.bfloat16),
        pltpu.VMEM((TB, Cc), jnp.float32),
        pltpu.SemaphoreType.DMA,
        pltpu.SemaphoreType.DMA,
    ]
    final, scores, eof, k = pl.pallas_call(
        _moe_step,
        grid=grid,
        in_specs=in_specs,
        out_specs=out_specs,
        out_shape=out_shapes,
        scratch_shapes=scratch_shapes,
        compiler_params=pltpu.CompilerParams(
            dimension_semantics=("arbitrary", "arbitrary"),
            vmem_limit_bytes=63 * 1024 * 1024,
        ),
    )(x32, sim_matrix, thr, w1, w2)

    return (final.reshape(Bb, Tt, Cc), scores, eof, k.reshape(N))


# speedup vs baseline: 1.9461x; 1.0042x over previous
import jax
import jax.numpy as jnp
from jax.experimental import pallas as pl
from jax.experimental.pallas import tpu as pltpu


def _moe_step(x_ref, sim_ref, thr_ref, w1_hbm, w2_hbm,
              final_ref, scores_ref, eof_hbm, k_ref,
              stag1, stag2, wb1, wb2, eofscr, xbs, sem_w, sem_o):
    e = pl.program_id(0)
    n = pl.program_id(1)
    ne = pl.num_programs(0)
    nb = pl.num_programs(1)
    tb = eofscr.shape[0]
    thr = thr_ref[0, 0]
    step = e * nb + n

    @pl.when(step == 0)
    def _():
        xf = x_ref[...]
        xn = xf / (jnp.sqrt(jnp.sum(xf * xf, axis=1, keepdims=True)) + 1e-12)
        sm = sim_ref[...]
        wn = sm / (jnp.sqrt(jnp.sum(sm * sm, axis=0, keepdims=True)) + 1e-12)
        s = jnp.dot(xn, wn, preferred_element_type=jnp.float32)
        scores_ref[...] = s
        k_ref[...] = jnp.sum((s > thr).astype(jnp.int32), axis=1, keepdims=True)
        xbs[...] = xf.astype(jnp.bfloat16)
        pltpu.make_async_copy(w1_hbm.at[0], stag1, sem_w).start()
        pltpu.make_async_copy(w2_hbm.at[0], stag2, sem_w).start()
        pltpu.make_async_copy(w1_hbm.at[0], stag1, sem_w).wait()
        pltpu.make_async_copy(w2_hbm.at[0], stag2, sem_w).wait()

    @pl.when(n == 0)
    def _():
        slot = jax.lax.rem(e, 2)

        @pl.when(e > 0)
        def _():
            pltpu.make_async_copy(w1_hbm.at[e], stag1, sem_w).wait()
            pltpu.make_async_copy(w2_hbm.at[e], stag2, sem_w).wait()

        wb1[slot] = stag1[...].astype(jnp.bfloat16)
        wb2[slot] = stag2[...].astype(jnp.bfloat16)

        @pl.when(e + 1 < ne)
        def _():
            pltpu.make_async_copy(w1_hbm.at[e + 1], stag1, sem_w).start()
            pltpu.make_async_copy(w2_hbm.at[e + 1], stag2, sem_w).start()

    slot = jax.lax.rem(e, 2)
    rows = pl.ds(n * tb, tb)
    xb = xbs[rows, :]
    h = jax.lax.dot_general(xb, wb1[slot], (((1,), (1,)), ((), ())),
                            preferred_element_type=jnp.float32)
    g = 0.5 * h * (1.0 + jax.lax.erf(h * 0.7071067811865476))
    out = jax.lax.dot_general(g.astype(jnp.bfloat16), wb2[slot],
                              (((1,), (1,)), ((), ())),
                              preferred_element_type=jnp.float32)
    s_blk = scores_ref[rows, :]
    onehot = (jax.lax.broadcasted_iota(jnp.int32, s_blk.shape, 1) == e)
    mask_col = jnp.sum(
        jnp.where((s_blk > thr) & onehot, 1.0, 0.0), axis=1, keepdims=True)
    mo = out * mask_col

    @pl.when(e == 0)
    def _():
        final_ref[rows, :] = mo

    @pl.when(e != 0)
    def _():
        final_ref[rows, :] = final_ref[rows, :] + mo

    @pl.when(step > 0)
    def _():
        sp = step - 1
        ep = sp // nb
        np_ = jax.lax.rem(sp, nb)
        prev_dst = eof_hbm.at[pl.ds(np_ * tb, tb), ep, :]
        pltpu.make_async_copy(eofscr, prev_dst, sem_o).wait()

    eofscr[...] = mo
    dst = eof_hbm.at[rows, e, :]
    pltpu.make_async_copy(eofscr, dst, sem_o).start()

    @pl.when(step == ne * nb - 1)
    def _():
        pltpu.make_async_copy(eofscr, dst, sem_o).wait()


def kernel(hidden_states, sim_matrix, threshold, w1, w2):
    Bb, Tt, Cc = hidden_states.shape
    Ee, Dff, _ = w1.shape
    N = Bb * Tt
    TB = 512
    NB = N // TB

    x32 = hidden_states.reshape(N, Cc)
    thr = threshold.reshape(1, 1)

    grid = (Ee, NB)
    out_shapes = (
        jax.ShapeDtypeStruct((N, Cc), jnp.float32),
        jax.ShapeDtypeStruct((N, Ee), jnp.float32),
        jax.ShapeDtypeStruct((N, Ee, Cc), jnp.float32),
        jax.ShapeDtypeStruct((N, 1), jnp.int32),
    )
    in_specs = [
        pl.BlockSpec((N, Cc), lambda e, n: (0, 0)),
        pl.BlockSpec((Cc, Ee), lambda e, n: (0, 0)),
        pl.BlockSpec((1, 1), lambda e, n: (0, 0)),
        pl.BlockSpec(memory_space=pltpu.MemorySpace.HBM),
        pl.BlockSpec(memory_space=pltpu.MemorySpace.HBM),
    ]
    out_specs = (
        pl.BlockSpec((N, Cc), lambda e, n: (0, 0)),
        pl.BlockSpec((N, Ee), lambda e, n: (0, 0)),
        pl.BlockSpec(memory_space=pltpu.MemorySpace.HBM),
        pl.BlockSpec((N, 1), lambda e, n: (0, 0)),
    )
    scratch_shapes = [
        pltpu.VMEM((Dff, Cc), jnp.float32),
        pltpu.VMEM((Cc, Dff), jnp.float32),
        pltpu.VMEM((2, Dff, Cc), jnp.bfloat16),
        pltpu.VMEM((2, Cc, Dff), jnp.bfloat16),
        pltpu.VMEM((TB, Cc), jnp.float32),
        pltpu.VMEM((N, Cc), jnp.bfloat16),
        pltpu.SemaphoreType.DMA,
        pltpu.SemaphoreType.DMA,
    ]
    final, scores, eof, k = pl.pallas_call(
        _moe_step,
        grid=grid,
        in_specs=in_specs,
        out_specs=out_specs,
        out_shape=out_shapes,
        scratch_shapes=scratch_shapes,
        compiler_params=pltpu.CompilerParams(
            dimension_semantics=("arbitrary", "arbitrary"),
            vmem_limit_bytes=67000000,
        ),
    )(x32, sim_matrix, thr, w1, w2)

    return (final.reshape(Bb, Tt, Cc), scores, eof, k.reshape(N))
